# baseline (speedup 1.0000x reference)
import jax
import jax.numpy as jnp
from jax.experimental import pallas as pl
from jax.experimental.pallas import tpu as pltpu

T = 8192
D_MODEL = 2048
N_EXPERTS = 16
TILE_T = 1024


def _ident(n):
    r = jax.lax.broadcasted_iota(jnp.int32, (n, n), 0)
    c = jax.lax.broadcasted_iota(jnp.int32, (n, n), 1)
    return (r == c).astype(jnp.float32)


def _routing_body(alpha_ref, hidden_ref, logits_ref, eg_ref, out_ref):
    alpha = alpha_ref[0, 0]
    sim = jax.lax.dot_general(
        hidden_ref[...], eg_ref[...], (((1,), (1,)), ((), ())),
        preferred_element_type=jnp.float32,
    )
    grounded = logits_ref[...] + alpha * sim

    g = jax.lax.dot_general(
        _ident(N_EXPERTS), grounded, (((1,), (1,)), ((), ())),
        precision=jax.lax.Precision.HIGHEST,
        preferred_element_type=jnp.float32,
    )

    idx = jax.lax.broadcasted_iota(jnp.int32, g.shape, 0)
    neg_inf = jnp.float32(-jnp.inf)

    m1 = jnp.max(g, axis=0, keepdims=True)
    i1 = jnp.min(jnp.where(g == m1, idx, N_EXPERTS), axis=0, keepdims=True)
    g2 = jnp.where(idx == i1, neg_inf, g)
    m2 = jnp.max(g2, axis=0, keepdims=True)
    i2 = jnp.min(jnp.where(g2 == m2, idx, N_EXPERTS), axis=0, keepdims=True)

    e = jnp.exp(m2 - m1)
    r = 1.0 / (1.0 + e)
    w1 = r
    w2 = e * r

    packed_t = jnp.concatenate(
        [i1.astype(jnp.float32), w1, i2.astype(jnp.float32), w2], axis=0
    )
    out_ref[...] = jax.lax.dot_general(
        packed_t, _ident(4), (((0,), (0,)), ((), ())),
        precision=jax.lax.Precision.HIGHEST,
        preferred_element_type=jnp.float32,
    )


@jax.jit
def _run(token_hidden, router_logits, expert_ground, alpha):
    alpha_arr = jnp.reshape(alpha.astype(jnp.float32), (1, 1))
    packed = pl.pallas_call(
        _routing_body,
        grid=(T // TILE_T,),
        in_specs=[
            pl.BlockSpec(memory_space=pltpu.SMEM),
            pl.BlockSpec((TILE_T, D_MODEL), lambda i: (i, 0)),
            pl.BlockSpec((TILE_T, N_EXPERTS), lambda i: (i, 0)),
            pl.BlockSpec((N_EXPERTS, D_MODEL), lambda i: (0, 0)),
        ],
        out_specs=pl.BlockSpec((TILE_T, 4), lambda i: (i, 0)),
        out_shape=jax.ShapeDtypeStruct((T, 4), jnp.float32),
        compiler_params=pltpu.CompilerParams(
            dimension_semantics=("arbitrary",),
        ),
    )(alpha_arr, token_hidden, router_logits, expert_ground)
    return packed.reshape(T, 2, 2)


def kernel(token_hidden, router_logits, expert_ground, alpha):
    return _run(token_hidden, router_logits, expert_ground, alpha)


# speedup vs baseline: 1.1504x; 1.1504x over previous
import jax
import jax.numpy as jnp
from jax.experimental import pallas as pl
from jax.experimental.pallas import tpu as pltpu

T = 8192
D_MODEL = 2048
N_EXPERTS = 16
CHUNK = 512
RING = 4
NCHUNKS = T // CHUNK


def _routing_chunk(c, alpha, h_buf, logits_ref, eg_ref, out_ref):
    sim = jax.lax.dot_general(
        h_buf[...], eg_ref[...], (((1,), (1,)), ((), ())),
        preferred_element_type=jnp.float32,
    )
    grounded = logits_ref[pl.ds(c * CHUNK, CHUNK), :] + alpha * sim
    g = jax.lax.transpose(grounded, (1, 0))

    idx = jax.lax.broadcasted_iota(jnp.int32, g.shape, 0)
    neg_inf = jnp.float32(-jnp.inf)

    m1 = jnp.max(g, axis=0, keepdims=True)
    i1 = jnp.min(jnp.where(g == m1, idx, N_EXPERTS), axis=0, keepdims=True)
    g2 = jnp.where(idx == i1, neg_inf, g)
    m2 = jnp.max(g2, axis=0, keepdims=True)
    i2 = jnp.min(jnp.where(g2 == m2, idx, N_EXPERTS), axis=0, keepdims=True)

    e = jnp.exp(m2 - m1)
    r = 1.0 / (1.0 + e)

    packed_t = jnp.concatenate(
        [i1.astype(jnp.float32), r, i2.astype(jnp.float32), e * r], axis=0
    )
    out_ref[pl.ds(c * CHUNK, CHUNK), :] = jax.lax.transpose(packed_t, (1, 0))


def _routing_body(alpha_ref, h_hbm, logits_ref, eg_ref, out_ref, *scratch):
    bufs = scratch[:RING]
    sems = scratch[RING]
    alpha = alpha_ref[0, 0]

    def h_copy(c):
        slot = c % RING
        return pltpu.make_async_copy(
            h_hbm.at[pl.ds(c * CHUNK, CHUNK), :], bufs[slot], sems.at[slot]
        )

    for c in range(RING):
        h_copy(c).start()
    for c in range(NCHUNKS):
        h_copy(c).wait()
        _routing_chunk(c, alpha, bufs[c % RING], logits_ref, eg_ref, out_ref)
        if c + RING < NCHUNKS:
            h_copy(c + RING).start()


@jax.jit
def _run(token_hidden, router_logits, expert_ground, alpha):
    alpha_arr = jnp.reshape(alpha.astype(jnp.float32), (1, 1))
    packed = pl.pallas_call(
        _routing_body,
        in_specs=[
            pl.BlockSpec(memory_space=pltpu.SMEM),
            pl.BlockSpec(memory_space=pl.ANY),
            pl.BlockSpec(memory_space=pltpu.VMEM),
            pl.BlockSpec(memory_space=pltpu.VMEM),
        ],
        out_specs=pl.BlockSpec(memory_space=pltpu.VMEM),
        out_shape=jax.ShapeDtypeStruct((T, 4), jnp.float32),
        scratch_shapes=(
            [pltpu.VMEM((CHUNK, D_MODEL), jnp.float32) for _ in range(RING)]
            + [pltpu.SemaphoreType.DMA((RING,))]
        ),
    )(alpha_arr, token_hidden, router_logits, expert_ground)
    return packed.reshape(T, 2, 2)


def kernel(token_hidden, router_logits, expert_ground, alpha):
    return _run(token_hidden, router_logits, expert_ground, alpha)


# speedup vs baseline: 1.2196x; 1.0601x over previous
import jax
import jax.numpy as jnp
from jax.experimental import pallas as pl
from jax.experimental.pallas import tpu as pltpu

T = 8192
D_MODEL = 2048
N_EXPERTS = 16
TILE_T = 1024
HALF = T // 2
GRID = HALF // TILE_T


def _top2_pack(grounded):
    g = jax.lax.transpose(grounded, (1, 0))

    idx = jax.lax.broadcasted_iota(jnp.int32, g.shape, 0)
    neg_inf = jnp.float32(-jnp.inf)

    m1 = jnp.max(g, axis=0, keepdims=True)
    i1 = jnp.min(jnp.where(g == m1, idx, N_EXPERTS), axis=0, keepdims=True)
    g2 = jnp.where(idx == i1, neg_inf, g)
    m2 = jnp.max(g2, axis=0, keepdims=True)
    i2 = jnp.min(jnp.where(g2 == m2, idx, N_EXPERTS), axis=0, keepdims=True)

    e = jnp.exp(m2 - m1)
    r = 1.0 / (1.0 + e)

    packed_t = jnp.concatenate(
        [i1.astype(jnp.float32), r, i2.astype(jnp.float32), e * r], axis=0
    )
    return jax.lax.transpose(packed_t, (1, 0))


def _routing_body(alpha_ref, ha_ref, hb_ref, la_ref, lb_ref, eg_ref, out_ref):
    alpha = alpha_ref[0, 0]
    dims = (((1,), (1,)), ((), ()))
    eg = eg_ref[...]
    sim_a = jax.lax.dot_general(
        ha_ref[...], eg, dims, preferred_element_type=jnp.float32
    )
    sim_b = jax.lax.dot_general(
        hb_ref[...], eg, dims, preferred_element_type=jnp.float32
    )
    out_ref[0] = _top2_pack(la_ref[...] + alpha * sim_a)
    out_ref[1] = _top2_pack(lb_ref[...] + alpha * sim_b)


def _row_spec(off):
    return pl.BlockSpec((TILE_T, D_MODEL), lambda i: (i + off, 0))


def _log_spec(off):
    return pl.BlockSpec((TILE_T, N_EXPERTS), lambda i: (i + off, 0))


@jax.jit
def _run(token_hidden, router_logits, expert_ground, alpha):
    alpha_arr = jnp.reshape(alpha.astype(jnp.float32), (1, 1))
    packed = pl.pallas_call(
        _routing_body,
        grid=(GRID,),
        in_specs=[
            pl.BlockSpec(memory_space=pltpu.SMEM),
            _row_spec(0),
            _row_spec(GRID),
            _log_spec(0),
            _log_spec(GRID),
            pl.BlockSpec((N_EXPERTS, D_MODEL), lambda i: (0, 0)),
        ],
        out_specs=pl.BlockSpec((2, TILE_T, 4), lambda i: (0, i, 0)),
        out_shape=jax.ShapeDtypeStruct((2, HALF, 4), jnp.float32),
        compiler_params=pltpu.CompilerParams(
            dimension_semantics=("arbitrary",),
        ),
    )(alpha_arr, token_hidden, token_hidden, router_logits, router_logits,
      expert_ground)
    return packed.reshape(T, 2, 2)


def kernel(token_hidden, router_logits, expert_ground, alpha):
    return _run(token_hidden, router_logits, expert_ground, alpha)


# speedup vs baseline: 1.3375x; 1.0967x over previous
import jax
import jax.numpy as jnp
from jax.experimental import pallas as pl
from jax.experimental.pallas import tpu as pltpu

T = 8192
D_MODEL = 2048
N_EXPERTS = 16
TILE_T = 1024


def _ident(n):
    r = jax.lax.broadcasted_iota(jnp.int32, (n, n), 0)
    c = jax.lax.broadcasted_iota(jnp.int32, (n, n), 1)
    return (r == c).astype(jnp.float32)


def _routing_body(alpha_ref, hidden_ref, logits_ref, eg_ref, out_ref):
    alpha = alpha_ref[0, 0]
    sim = jax.lax.dot_general(
        hidden_ref[...], eg_ref[...], (((1,), (1,)), ((), ())),
        preferred_element_type=jnp.float32,
    )
    grounded = logits_ref[...] + alpha * sim

    g = jax.lax.transpose(grounded, (1, 0))

    idx = jax.lax.broadcasted_iota(jnp.int32, g.shape, 0)
    neg_inf = jnp.float32(-jnp.inf)

    m1 = jnp.max(g, axis=0, keepdims=True)
    i1 = jnp.min(jnp.where(g == m1, idx, N_EXPERTS), axis=0, keepdims=True)
    g2 = jnp.where(idx == i1, neg_inf, g)
    m2 = jnp.max(g2, axis=0, keepdims=True)
    i2 = jnp.min(jnp.where(g2 == m2, idx, N_EXPERTS), axis=0, keepdims=True)

    e = jnp.exp(m2 - m1)
    r = 1.0 / (1.0 + e)
    w1 = r
    w2 = e * r

    packed_t = jnp.concatenate(
        [i1.astype(jnp.float32), w1, i2.astype(jnp.float32), w2], axis=0
    )
    out_ref[...] = jax.lax.transpose(packed_t, (1, 0))


@jax.jit
def _run(token_hidden, router_logits, expert_ground, alpha):
    alpha_arr = jnp.reshape(alpha.astype(jnp.float32), (1, 1))
    packed = pl.pallas_call(
        _routing_body,
        grid=(T // TILE_T,),
        in_specs=[
            pl.BlockSpec(memory_space=pltpu.SMEM),
            pl.BlockSpec((TILE_T, D_MODEL), lambda i: (i, 0)),
            pl.BlockSpec((TILE_T, N_EXPERTS), lambda i: (i, 0)),
            pl.BlockSpec((N_EXPERTS, D_MODEL), lambda i: (0, 0)),
        ],
        out_specs=pl.BlockSpec((TILE_T, 4), lambda i: (i, 0)),
        out_shape=jax.ShapeDtypeStruct((T, 4), jnp.float32),
        compiler_params=pltpu.CompilerParams(
            dimension_semantics=("parallel",),
        ),
    )(alpha_arr, token_hidden, router_logits, expert_ground)
    return packed.reshape(T, 2, 2)


def kernel(token_hidden, router_logits, expert_ground, alpha):
    return _run(token_hidden, router_logits, expert_ground, alpha)
